# baseline (speedup 1.0000x reference)
import functools

import jax
import jax.numpy as jnp
from jax import lax
from jax.experimental import pallas as pl
from jax.experimental.pallas import tpu as pltpu


def _qsample_body(t_ref, ab_ref, x0_ref, noise_ref, out_ref):
    b = pl.program_id(0)
    ab = ab_ref[t_ref[b]]
    sa = jnp.sqrt(ab)
    sb = jnp.sqrt(1.0 - ab)
    out_ref[...] = sa * x0_ref[...] + sb * noise_ref[...]


@jax.jit
def kernel(x0, t, noise, alpha_bars):
    B, L, D = x0.shape
    BL = 512
    grid = (B, L // BL)
    blk = pl.BlockSpec((1, BL, D), lambda b, l: (b, l, 0))
    return pl.pallas_call(
        _qsample_body,
        grid=grid,
        in_specs=[
            pl.BlockSpec(memory_space=pltpu.SMEM),
            pl.BlockSpec(memory_space=pltpu.SMEM),
            blk,
            blk,
        ],
        out_specs=blk,
        out_shape=jax.ShapeDtypeStruct((B, L, D), jnp.float32),
        compiler_params=pltpu.CompilerParams(
            dimension_semantics=("parallel", "parallel"),
        ),
    )(t, alpha_bars, x0, noise)


# speedup vs baseline: 2.1988x; 2.1988x over previous
import functools

import jax
import jax.numpy as jnp
from jax import lax
from jax.experimental import pallas as pl
from jax.experimental.pallas import tpu as pltpu


def _qsample_body(t_ref, ab_ref, x0_ref, noise_ref, out_ref):
    b = pl.program_id(0)
    ab = ab_ref[t_ref[b]]
    sa = jnp.sqrt(ab)
    sb = jnp.sqrt(1.0 - ab)
    out_ref[...] = sa * x0_ref[...] + sb * noise_ref[...]


@jax.jit
def kernel(x0, t, noise, alpha_bars):
    B, L, D = x0.shape
    BL = 2048
    grid = (B, L // BL)
    blk = pl.BlockSpec((1, BL, D), lambda b, l: (b, l, 0))
    return pl.pallas_call(
        _qsample_body,
        grid=grid,
        in_specs=[
            pl.BlockSpec(memory_space=pltpu.SMEM),
            pl.BlockSpec(memory_space=pltpu.SMEM),
            blk,
            blk,
        ],
        out_specs=blk,
        out_shape=jax.ShapeDtypeStruct((B, L, D), jnp.float32),
        compiler_params=pltpu.CompilerParams(
            dimension_semantics=("parallel", "parallel"),
        ),
    )(t, alpha_bars, x0, noise)


# speedup vs baseline: 2.7158x; 1.2351x over previous
import functools

import jax
import jax.numpy as jnp
from jax import lax
from jax.experimental import pallas as pl
from jax.experimental.pallas import tpu as pltpu


def _qsample_body(t_ref, ab_ref, x0_ref, noise_ref, out_ref, *, nb):
    g = pl.program_id(0)
    for j in range(nb):
        b = g * nb + j
        ab = ab_ref[t_ref[b]]
        sa = jnp.sqrt(ab)
        sb = jnp.sqrt(1.0 - ab)
        out_ref[j] = sa * x0_ref[j] + sb * noise_ref[j]


@jax.jit
def kernel(x0, t, noise, alpha_bars):
    B, L, D = x0.shape
    NB = 2
    grid = (B // NB,)
    blk = pl.BlockSpec((NB, L, D), lambda g: (g, 0, 0))
    return pl.pallas_call(
        functools.partial(_qsample_body, nb=NB),
        grid=grid,
        in_specs=[
            pl.BlockSpec(memory_space=pltpu.SMEM),
            pl.BlockSpec(memory_space=pltpu.SMEM),
            blk,
            blk,
        ],
        out_specs=blk,
        out_shape=jax.ShapeDtypeStruct((B, L, D), jnp.float32),
        compiler_params=pltpu.CompilerParams(
            dimension_semantics=("parallel",),
        ),
    )(t, alpha_bars, x0, noise)


# speedup vs baseline: 2.8202x; 1.0385x over previous
import functools

import jax
import jax.numpy as jnp
from jax import lax
from jax.experimental import pallas as pl
from jax.experimental.pallas import tpu as pltpu


def _qsample_body(t_ref, ab_ref, x0_ref, noise_ref, out_ref, *, nb):
    g = pl.program_id(0)
    for j in range(nb):
        b = g * nb + j
        ab = ab_ref[t_ref[b]]
        sa = jnp.sqrt(ab)
        sb = jnp.sqrt(1.0 - ab)
        out_ref[j] = sa * x0_ref[j] + sb * noise_ref[j]


@jax.jit
def kernel(x0, t, noise, alpha_bars):
    B, L, D = x0.shape
    NB = 4
    grid = (B // NB,)
    blk = pl.BlockSpec((NB, L, D), lambda g: (g, 0, 0))
    return pl.pallas_call(
        functools.partial(_qsample_body, nb=NB),
        grid=grid,
        in_specs=[
            pl.BlockSpec(memory_space=pltpu.SMEM),
            pl.BlockSpec(memory_space=pltpu.SMEM),
            blk,
            blk,
        ],
        out_specs=blk,
        out_shape=jax.ShapeDtypeStruct((B, L, D), jnp.float32),
        compiler_params=pltpu.CompilerParams(
            dimension_semantics=("parallel",),
        ),
    )(t, alpha_bars, x0, noise)
